# Initial kernel scaffold; baseline (speedup 1.0000x reference)
import functools

import jax
import jax.numpy as jnp
from jax import lax
from jax.experimental import pallas as pl
from jax.experimental.pallas import tpu as pltpu
from jax.experimental.pallas import tpu_sc as plsc

N = 10000
E = 320000
D = 128
H1, C1 = 8, 16
C2 = 16

NC, NS = 2, 16
NW = NC * NS
EPT = E // NW
B = 80
NCH = EPT // B
STRIPE = N // NS


def _contract(W, att, heads, ch):
    return jnp.einsum('dhc,hc->dh', W.reshape(W.shape[0], heads, ch), att[0])


def _node_mm_body(x_ref, w_ref, t_ref, ad_ref, *, split):
    out = jnp.dot(x_ref[...], w_ref[...], preferred_element_type=jnp.float32)
    t_ref[...] = out[:, :split]
    ad_ref[...] = out[:, split:]


def _node_tables(x, wcat, split):
    rows, k = x.shape
    w = wcat.shape[1]
    blk = 1000
    return pl.pallas_call(
        functools.partial(_node_mm_body, split=split),
        grid=(rows // blk,),
        in_specs=[
            pl.BlockSpec((blk, k), lambda i: (i, 0)),
            pl.BlockSpec((k, w), lambda i: (0, 0)),
        ],
        out_specs=[
            pl.BlockSpec((blk, split), lambda i: (i, 0)),
            pl.BlockSpec((blk, w - split), lambda i: (i, 0)),
        ],
        out_shape=[
            jax.ShapeDtypeStruct((rows, split), jnp.float32),
            jax.ShapeDtypeStruct((rows, w - split), jnp.float32),
        ],
    )(x, wcat)


def _edge_mm_body(ea_ref, m_ref, a1_ref, a2_ref):
    out = jnp.dot(ea_ref[...], m_ref[...], preferred_element_type=jnp.float32)
    a1_ref[...] = out[:, :16]
    a2_ref[...] = out[:, 16:]


def _edge_tables(edge_attr, mcat):
    blk = 8000
    return pl.pallas_call(
        _edge_mm_body,
        grid=(E // blk,),
        in_specs=[
            pl.BlockSpec((blk, 16), lambda i: (i, 0)),
            pl.BlockSpec((16, 32), lambda i: (0, 0)),
        ],
        out_specs=[
            pl.BlockSpec((blk, 16), lambda i: (i, 0)),
            pl.BlockSpec((blk, 16), lambda i: (i, 0)),
        ],
        out_shape=[
            jax.ShapeDtypeStruct((E, 16), jnp.float32),
            jax.ShapeDtypeStruct((E, 16), jnp.float32),
        ],
    )(edge_attr, mcat)


def _layer1_finish_body(accp_ref, b1_ref, w_ref, t2_ref, ad2_ref):
    acc = accp_ref[0] + accp_ref[1]
    num = acc[:, :128].reshape(-1, H1, C1)
    den = acc[:, 128:136].reshape(-1, H1, 1)
    v = (num / (den + 1e-16)).reshape(-1, 128) + b1_ref[0]
    h1 = jnp.where(v > 0, v, jnp.exp(v) - 1.0)
    out = jnp.dot(h1, w_ref[...], preferred_element_type=jnp.float32)
    t2_ref[...] = out[:, :32]
    ad2_ref[...] = out[:, 32:]


def _layer1_finish(accp1, b1, wcat2):
    blk = 1000
    return pl.pallas_call(
        _layer1_finish_body,
        grid=(N // blk,),
        in_specs=[
            pl.BlockSpec((2, blk, 144), lambda i: (0, i, 0)),
            pl.BlockSpec((1, 128), lambda i: (0, 0)),
            pl.BlockSpec((128, 48), lambda i: (0, 0)),
        ],
        out_specs=[
            pl.BlockSpec((blk, 32), lambda i: (i, 0)),
            pl.BlockSpec((blk, 16), lambda i: (i, 0)),
        ],
        out_shape=[
            jax.ShapeDtypeStruct((N, 32), jnp.float32),
            jax.ShapeDtypeStruct((N, 16), jnp.float32),
        ],
    )(accp1, b1.reshape(1, 128), wcat2)


def _final_body(accp_ref, b2_ref, o_ref):
    acc = accp_ref[0] + accp_ref[1]
    z = acc[:, :16] / (acc[:, 16:17] + 1e-16) + b2_ref[0]
    m = jnp.max(z, axis=1, keepdims=True)
    zz = z - m
    lse = jnp.log(jnp.sum(jnp.exp(zz), axis=1, keepdims=True))
    o_ref[...] = zz - lse


def _final(accp2, b2):
    blk = 1000
    return pl.pallas_call(
        _final_body,
        grid=(N // blk,),
        in_specs=[
            pl.BlockSpec((2, blk, 32), lambda i: (0, i, 0)),
            pl.BlockSpec((1, 16), lambda i: (0, 0)),
        ],
        out_specs=pl.BlockSpec((blk, 16), lambda i: (i, 0)),
        out_shape=jax.ShapeDtypeStruct((N, 16), jnp.float32),
    )(accp2, b2.reshape(1, 16))


def _edge_phase(table, adst, src2d, dst2d, ae, zeros, width, logit_off, nheads):
    mesh = plsc.VectorSubcoreMesh(core_axis_name="c", subcore_axis_name="s")

    @functools.partial(
        pl.kernel,
        out_type=jax.ShapeDtypeStruct((NC, N, width), jnp.float32),
        mesh=mesh,
        scratch_types=[
            pltpu.VMEM((1, B), jnp.int32),
            pltpu.VMEM((1, B), jnp.int32),
            pltpu.VMEM((B, 16), jnp.float32),
            pltpu.VMEM((B, width), jnp.float32),
            pltpu.VMEM((B, 16), jnp.float32),
            pltpu.VMEM((B, width), jnp.float32),
            pltpu.VMEM_SHARED((N, width), jnp.float32),
        ],
    )
    def k(tab, ad, srcr, dstr, aer, zr, accp,
          sidx, didx, aec, srows, drows, msg, shacc):
        c = lax.axis_index("c")
        s = lax.axis_index("s")
        pltpu.sync_copy(zr.at[pl.ds(s * STRIPE, STRIPE)],
                        shacc.at[pl.ds(s * STRIPE, STRIPE)])
        plsc.subcore_barrier()
        base0 = (c * NS + s) * EPT

        @pl.loop(0, NCH)
        def _(i):
            base = base0 + i * B
            pltpu.sync_copy(srcr.at[:, pl.ds(base, B)], sidx)
            pltpu.sync_copy(dstr.at[:, pl.ds(base, B)], didx)
            pltpu.sync_copy(aer.at[pl.ds(base, B)], aec)
            pltpu.sync_copy(tab.at[sidx.at[0]], srows)
            pltpu.sync_copy(ad.at[didx.at[0]], drows)

            @pl.loop(0, B)
            def _(e):
                logit = srows[e, pl.ds(logit_off, 16)] + drows[e, :] + aec[e, :]
                l = jnp.where(logit > 0, logit, logit * 0.2)
                msg[e, pl.ds(logit_off, 16)] = jnp.exp(l)
                for h in range(nheads):
                    w = msg[e, logit_off + h]
                    msg[e, pl.ds(h * 16, 16)] = srows[e, pl.ds(h * 16, 16)] * w

            pltpu.sync_copy(msg, shacc.at[didx.at[0]], add=True)

        plsc.subcore_barrier()
        pltpu.sync_copy(shacc.at[pl.ds(s * STRIPE, STRIPE)],
                        accp.at[c, pl.ds(s * STRIPE, STRIPE)])

    return k(table, adst, src2d, dst2d, ae, zeros)


def kernel(x, edge_index, edge_attr, W1, att_src1, att_dst1, We1, att_edge1,
           b1, W2, att_src2, att_dst2, We2, att_edge2, b2):
    f32 = jnp.float32
    Wsrc1 = _contract(W1, att_src1, H1, C1)
    Wdst1 = _contract(W1, att_dst1, H1, C1)
    Me1 = _contract(We1, att_edge1, H1, C1)
    Wsrc2 = _contract(W2, att_src2, 1, C2)
    Wdst2 = _contract(W2, att_dst2, 1, C2)
    Me2 = _contract(We2, att_edge2, 1, C2)
    z8 = jnp.zeros((D, 8), f32)
    wcat1 = jnp.concatenate([W1, Wsrc1, z8, Wdst1, z8], axis=1)
    mcat = jnp.concatenate([Me1, jnp.zeros((16, 8), f32),
                            Me2, jnp.zeros((16, 15), f32)], axis=1)
    z15 = jnp.zeros((D, 15), f32)
    wcat2 = jnp.concatenate([W2, Wsrc2, z15, Wdst2, z15], axis=1)

    src2d = edge_index[0].reshape(1, E).astype(jnp.int32)
    dst2d = edge_index[1].reshape(1, E).astype(jnp.int32)

    table1, adst1 = _node_tables(x, wcat1, 144)
    ae1, ae2 = _edge_tables(edge_attr, mcat)

    zeros1 = jnp.zeros((N, 144), f32)
    accp1 = _edge_phase(table1, adst1, src2d, dst2d, ae1, zeros1,
                        width=144, logit_off=128, nheads=8)

    table2, adst2 = _layer1_finish(accp1, b1, wcat2)

    zeros2 = jnp.zeros((N, 32), f32)
    accp2 = _edge_phase(table2, adst2, src2d, dst2d, ae2, zeros2,
                        width=32, logit_off=16, nheads=1)

    return _final(accp2, b2)


# speedup vs baseline: 24.7014x; 24.7014x over previous
import functools

import jax
import jax.numpy as jnp
from jax import lax
from jax.experimental import pallas as pl
from jax.experimental.pallas import tpu as pltpu
from jax.experimental.pallas import tpu_sc as plsc

N = 10000
E = 320000
D = 128
H1, C1 = 8, 16
C2 = 16

NC, NS = 2, 16
NW = NC * NS
B = 80
NCHUNKS = E // B
ROUNDS = NCHUNKS // NW
REM = NCHUNKS - ROUNDS * NW
STRIPE = 624
TAIL = N - NS * STRIPE
TAIL_OFF = NS * STRIPE


def _contract(W, att, heads, ch):
    return jnp.einsum('dhc,hc->dh', W.reshape(W.shape[0], heads, ch), att[0])


def _node_mm_body(x_ref, w_ref, t_ref, ad_ref, *, split):
    out = jnp.dot(x_ref[...], w_ref[...], preferred_element_type=jnp.float32)
    t_ref[...] = out[:, :split]
    ad_ref[...] = out[:, split:]


def _node_tables(x, wcat, split):
    rows, k = x.shape
    w = wcat.shape[1]
    blk = 1000
    return pl.pallas_call(
        functools.partial(_node_mm_body, split=split),
        grid=(rows // blk,),
        in_specs=[
            pl.BlockSpec((blk, k), lambda i: (i, 0)),
            pl.BlockSpec((k, w), lambda i: (0, 0)),
        ],
        out_specs=[
            pl.BlockSpec((blk, split), lambda i: (i, 0)),
            pl.BlockSpec((blk, w - split), lambda i: (i, 0)),
        ],
        out_shape=[
            jax.ShapeDtypeStruct((rows, split), jnp.float32),
            jax.ShapeDtypeStruct((rows, w - split), jnp.float32),
        ],
    )(x, wcat)


def _edge_mm_body(ea_ref, m_ref, a1_ref, a2_ref):
    out = jnp.dot(ea_ref[...], m_ref[...], preferred_element_type=jnp.float32)
    a1_ref[...] = out[:, :16]
    a2_ref[...] = out[:, 16:]


def _edge_tables(edge_attr, mcat):
    blk = 8000
    return pl.pallas_call(
        _edge_mm_body,
        grid=(E // blk,),
        in_specs=[
            pl.BlockSpec((blk, 16), lambda i: (i, 0)),
            pl.BlockSpec((16, 32), lambda i: (0, 0)),
        ],
        out_specs=[
            pl.BlockSpec((blk, 16), lambda i: (i, 0)),
            pl.BlockSpec((blk, 16), lambda i: (i, 0)),
        ],
        out_shape=[
            jax.ShapeDtypeStruct((E, 16), jnp.float32),
            jax.ShapeDtypeStruct((E, 16), jnp.float32),
        ],
    )(edge_attr, mcat)


def _layer1_finish_body(accp_ref, b1_ref, w_ref, t2_ref, ad2_ref):
    acc = accp_ref[0] + accp_ref[1]
    num = acc[:, :128].reshape(-1, H1, C1)
    den = acc[:, 128:136].reshape(-1, H1, 1)
    v = (num / (den + 1e-16)).reshape(-1, 128) + b1_ref[0]
    h1 = jnp.where(v > 0, v, jnp.exp(v) - 1.0)
    out = jnp.dot(h1, w_ref[...], preferred_element_type=jnp.float32)
    t2_ref[...] = out[:, :32]
    ad2_ref[...] = out[:, 32:]


def _layer1_finish(accp1, b1, wcat2):
    blk = 1000
    return pl.pallas_call(
        _layer1_finish_body,
        grid=(N // blk,),
        in_specs=[
            pl.BlockSpec((2, blk, 144), lambda i: (0, i, 0)),
            pl.BlockSpec((1, 128), lambda i: (0, 0)),
            pl.BlockSpec((128, 48), lambda i: (0, 0)),
        ],
        out_specs=[
            pl.BlockSpec((blk, 32), lambda i: (i, 0)),
            pl.BlockSpec((blk, 16), lambda i: (i, 0)),
        ],
        out_shape=[
            jax.ShapeDtypeStruct((N, 32), jnp.float32),
            jax.ShapeDtypeStruct((N, 16), jnp.float32),
        ],
    )(accp1, b1.reshape(1, 128), wcat2)


def _final_body(accp_ref, b2_ref, o_ref):
    acc = accp_ref[0] + accp_ref[1]
    z = acc[:, :16] / (acc[:, 16:17] + 1e-16) + b2_ref[0]
    m = jnp.max(z, axis=1, keepdims=True)
    zz = z - m
    lse = jnp.log(jnp.sum(jnp.exp(zz), axis=1, keepdims=True))
    o_ref[...] = zz - lse


def _final(accp2, b2):
    blk = 1000
    return pl.pallas_call(
        _final_body,
        grid=(N // blk,),
        in_specs=[
            pl.BlockSpec((2, blk, 32), lambda i: (0, i, 0)),
            pl.BlockSpec((1, 16), lambda i: (0, 0)),
        ],
        out_specs=pl.BlockSpec((blk, 16), lambda i: (i, 0)),
        out_shape=jax.ShapeDtypeStruct((N, 16), jnp.float32),
    )(accp2, b2.reshape(1, 16))


def _edge_phase(table, adst, src2d, dst2d, ae, zeros, width, logit_off, nheads):
    mesh = plsc.VectorSubcoreMesh(core_axis_name="c", subcore_axis_name="s")

    @functools.partial(
        pl.kernel,
        out_type=jax.ShapeDtypeStruct((NC, N, width), jnp.float32),
        mesh=mesh,
        compiler_params=pltpu.CompilerParams(use_tc_tiling_on_sc=False,
                                             internal_scratch_in_bytes=1 << 16),
        scratch_types=[
            pltpu.VMEM((1, B), jnp.int32),
            pltpu.VMEM((1, B), jnp.int32),
            pltpu.VMEM((B, 16), jnp.float32),
            pltpu.VMEM((B, width), jnp.float32),
            pltpu.VMEM((B, 16), jnp.float32),
            pltpu.VMEM((B, width), jnp.float32),
            pltpu.VMEM_SHARED((N, width), jnp.float32),
        ],
    )
    def k(tab, ad, srcr, dstr, aer, zr, accp,
          sidx, didx, aec, srows, drows, msg, shacc):
        c = lax.axis_index("c")
        s = lax.axis_index("s")
        pltpu.sync_copy(zr.at[pl.ds(s * STRIPE, STRIPE)],
                        shacc.at[pl.ds(s * STRIPE, STRIPE)])

        @pl.when(s == 0)
        def _():
            pltpu.sync_copy(zr.at[pl.ds(TAIL_OFF, TAIL)],
                            shacc.at[pl.ds(TAIL_OFF, TAIL)])

        plsc.subcore_barrier()
        tile = c * NS + s

        def do_chunk(base):
            pltpu.sync_copy(srcr.at[:, pl.ds(base, B)], sidx)
            pltpu.sync_copy(dstr.at[:, pl.ds(base, B)], didx)
            pltpu.sync_copy(aer.at[pl.ds(base, B)], aec)
            pltpu.sync_copy(tab.at[sidx.at[0]], srows)
            pltpu.sync_copy(ad.at[didx.at[0]], drows)

            @pl.loop(0, B)
            def _(e):
                logit = srows[e, pl.ds(logit_off, 16)] + drows[e, :] + aec[e, :]
                l = jnp.where(logit > 0, logit, logit * 0.2)
                ex = jnp.exp(l)
                msg[e, pl.ds(logit_off, 16)] = ex
                for h in range(nheads):
                    msg[e, pl.ds(h * 16, 16)] = srows[e, pl.ds(h * 16, 16)] * ex[h]

            pltpu.sync_copy(msg, shacc.at[didx.at[0]], add=True)

        @pl.loop(0, ROUNDS)
        def _(i):
            do_chunk((i * NW + tile) * B)

        if REM:
            @pl.when(tile < REM)
            def _():
                do_chunk((ROUNDS * NW + tile) * B)

        plsc.subcore_barrier()
        pltpu.sync_copy(shacc.at[pl.ds(s * STRIPE, STRIPE)],
                        accp.at[c, pl.ds(s * STRIPE, STRIPE)])

        @pl.when(s == 0)
        def _():
            pltpu.sync_copy(shacc.at[pl.ds(TAIL_OFF, TAIL)],
                            accp.at[c, pl.ds(TAIL_OFF, TAIL)])

    return k(table, adst, src2d, dst2d, ae, zeros)


def kernel(x, edge_index, edge_attr, W1, att_src1, att_dst1, We1, att_edge1,
           b1, W2, att_src2, att_dst2, We2, att_edge2, b2):
    f32 = jnp.float32
    Wsrc1 = _contract(W1, att_src1, H1, C1)
    Wdst1 = _contract(W1, att_dst1, H1, C1)
    Me1 = _contract(We1, att_edge1, H1, C1)
    Wsrc2 = _contract(W2, att_src2, 1, C2)
    Wdst2 = _contract(W2, att_dst2, 1, C2)
    Me2 = _contract(We2, att_edge2, 1, C2)
    z8 = jnp.zeros((D, 8), f32)
    wcat1 = jnp.concatenate([W1, Wsrc1, z8, Wdst1, z8], axis=1)
    mcat = jnp.concatenate([Me1, jnp.zeros((16, 8), f32),
                            Me2, jnp.zeros((16, 15), f32)], axis=1)
    z15 = jnp.zeros((D, 15), f32)
    wcat2 = jnp.concatenate([W2, Wsrc2, z15, Wdst2, z15], axis=1)

    src2d = edge_index[0].reshape(1, E).astype(jnp.int32)
    dst2d = edge_index[1].reshape(1, E).astype(jnp.int32)

    table1, adst1 = _node_tables(x, wcat1, 144)
    ae1, ae2 = _edge_tables(edge_attr, mcat)

    zeros1 = jnp.zeros((N, 144), f32)
    accp1 = _edge_phase(table1, adst1, src2d, dst2d, ae1, zeros1,
                        width=144, logit_off=128, nheads=8)

    table2, adst2 = _layer1_finish(accp1, b1, wcat2)

    zeros2 = jnp.zeros((N, 32), f32)
    accp2 = _edge_phase(table2, adst2, src2d, dst2d, ae2, zeros2,
                        width=32, logit_off=16, nheads=1)

    return _final(accp2, b2)


# speedup vs baseline: 39.8479x; 1.6132x over previous
import functools

import jax
import jax.numpy as jnp
from jax import lax
from jax.experimental import pallas as pl
from jax.experimental.pallas import tpu as pltpu
from jax.experimental.pallas import tpu_sc as plsc

N = 10000
E = 320000
D = 128
H1, C1 = 8, 16
C2 = 16

NC, NS = 2, 16
NW = NC * NS
B = 80
NCHUNKS = E // B
NCH = NCHUNKS // NW
CPS = 5
STRIPE = 624
TAIL = N - NS * STRIPE
TAIL_OFF = NS * STRIPE


def _contract(W, att, heads, ch):
    return jnp.einsum('dhc,hc->dh', W.reshape(W.shape[0], heads, ch), att[0])


def _node_mm_body(x_ref, w_ref, t_ref, ad_ref, *, split):
    out = jnp.dot(x_ref[...], w_ref[...], preferred_element_type=jnp.float32)
    t_ref[...] = out[:, :split]
    ad_ref[...] = out[:, split:]


def _node_tables(x, wcat, split):
    rows, k = x.shape
    w = wcat.shape[1]
    blk = 1000
    return pl.pallas_call(
        functools.partial(_node_mm_body, split=split),
        grid=(rows // blk,),
        in_specs=[
            pl.BlockSpec((blk, k), lambda i: (i, 0)),
            pl.BlockSpec((k, w), lambda i: (0, 0)),
        ],
        out_specs=[
            pl.BlockSpec((blk, split), lambda i: (i, 0)),
            pl.BlockSpec((blk, w - split), lambda i: (i, 0)),
        ],
        out_shape=[
            jax.ShapeDtypeStruct((rows, split), jnp.float32),
            jax.ShapeDtypeStruct((rows, w - split), jnp.float32),
        ],
    )(x, wcat)


def _edge_mm_body(ea_ref, m_ref, a1_ref, a2_ref):
    out = jnp.dot(ea_ref[...], m_ref[...], preferred_element_type=jnp.float32)
    a1_ref[...] = out[:, :16]
    a2_ref[...] = out[:, 16:]


def _edge_tables(edge_attr, mcat):
    blk = 8000
    return pl.pallas_call(
        _edge_mm_body,
        grid=(E // blk,),
        in_specs=[
            pl.BlockSpec((blk, 16), lambda i: (i, 0)),
            pl.BlockSpec((16, 32), lambda i: (0, 0)),
        ],
        out_specs=[
            pl.BlockSpec((blk, 16), lambda i: (i, 0)),
            pl.BlockSpec((blk, 16), lambda i: (i, 0)),
        ],
        out_shape=[
            jax.ShapeDtypeStruct((E, 16), jnp.float32),
            jax.ShapeDtypeStruct((E, 16), jnp.float32),
        ],
    )(edge_attr, mcat)


def _layer1_finish_body(accp_ref, b1_ref, w_ref, t2_ref, ad2_ref):
    acc = accp_ref[0] + accp_ref[1]
    num = acc[:, :128].reshape(-1, H1, C1)
    den = acc[:, 128:136].reshape(-1, H1, 1)
    v = (num / (den + 1e-16)).reshape(-1, 128) + b1_ref[0]
    h1 = jnp.where(v > 0, v, jnp.exp(v) - 1.0)
    out = jnp.dot(h1, w_ref[...], preferred_element_type=jnp.float32)
    t2_ref[...] = out[:, :32]
    ad2_ref[...] = out[:, 32:]


def _layer1_finish(accp1, b1, wcat2):
    blk = 1000
    return pl.pallas_call(
        _layer1_finish_body,
        grid=(N // blk,),
        in_specs=[
            pl.BlockSpec((2, blk, 144), lambda i: (0, i, 0)),
            pl.BlockSpec((1, 128), lambda i: (0, 0)),
            pl.BlockSpec((128, 48), lambda i: (0, 0)),
        ],
        out_specs=[
            pl.BlockSpec((blk, 32), lambda i: (i, 0)),
            pl.BlockSpec((blk, 16), lambda i: (i, 0)),
        ],
        out_shape=[
            jax.ShapeDtypeStruct((N, 32), jnp.float32),
            jax.ShapeDtypeStruct((N, 16), jnp.float32),
        ],
    )(accp1, b1.reshape(1, 128), wcat2)


def _final_body(accp_ref, b2_ref, o_ref):
    acc = accp_ref[0] + accp_ref[1]
    z = acc[:, :16] / (acc[:, 16:17] + 1e-16) + b2_ref[0]
    m = jnp.max(z, axis=1, keepdims=True)
    zz = z - m
    lse = jnp.log(jnp.sum(jnp.exp(zz), axis=1, keepdims=True))
    o_ref[...] = zz - lse


def _final(accp2, b2):
    blk = 1000
    return pl.pallas_call(
        _final_body,
        grid=(N // blk,),
        in_specs=[
            pl.BlockSpec((2, blk, 32), lambda i: (0, i, 0)),
            pl.BlockSpec((1, 16), lambda i: (0, 0)),
        ],
        out_specs=pl.BlockSpec((blk, 16), lambda i: (i, 0)),
        out_shape=jax.ShapeDtypeStruct((N, 16), jnp.float32),
    )(accp2, b2.reshape(1, 16))


def _edge_phase(table, adst, eidx3, ae3, zeros, width, logit_off, nheads):
    mesh = plsc.VectorSubcoreMesh(core_axis_name="c", subcore_axis_name="s")

    @functools.partial(
        pl.kernel,
        out_type=jax.ShapeDtypeStruct((NC, N, width), jnp.float32),
        mesh=mesh,
        compiler_params=pltpu.CompilerParams(use_tc_tiling_on_sc=False),
        scratch_types=[
            pltpu.VMEM((2, CPS, B), jnp.int32),
            pltpu.VMEM((CPS, B, 16), jnp.float32),
            pltpu.VMEM((B, width), jnp.float32),
            pltpu.VMEM((B, width), jnp.float32),
            pltpu.VMEM((B, 16), jnp.float32),
            pltpu.VMEM((B, 16), jnp.float32),
            pltpu.VMEM_SHARED((N, width), jnp.float32),
            pltpu.SemaphoreType.DMA,
            pltpu.SemaphoreType.DMA,
        ],
    )
    def k(tab, ad, eidx, aer, zr, accp,
          scidx, scae, rows0, rows1, d0, d1, shacc, sem0, sem1):
        c = lax.axis_index("c")
        s = lax.axis_index("s")
        pltpu.sync_copy(zr.at[pl.ds(s * STRIPE, STRIPE)],
                        shacc.at[pl.ds(s * STRIPE, STRIPE)])

        @pl.when(s == 0)
        def _():
            pltpu.sync_copy(zr.at[pl.ds(TAIL_OFF, TAIL)],
                            shacc.at[pl.ds(TAIL_OFF, TAIL)])

        plsc.subcore_barrier()
        tile = c * NS + s
        chunk0 = tile * NCH

        def copy_superchunk(sc):
            g = chunk0 + sc * CPS
            pltpu.sync_copy(eidx.at[:, pl.ds(g, CPS)], scidx)
            pltpu.sync_copy(aer.at[pl.ds(g, CPS)], scae)

        def start_gather(kk, rows, d, sem):
            m = lax.rem(kk, CPS)
            pltpu.async_copy(tab.at[scidx.at[0, m]], rows, sem)
            pltpu.async_copy(ad.at[scidx.at[1, m]], d, sem)

        def wait_gather(rows, d, sem):
            pltpu.make_async_copy(tab.at[scidx.at[0, 0]], rows, sem).wait()
            pltpu.make_async_copy(ad.at[scidx.at[1, 0]], d, sem).wait()

        def compute_scatter(kk, rows, d):
            m = lax.rem(kk, CPS)

            @pl.loop(0, B)
            def _(e):
                logit = rows[e, pl.ds(logit_off, 16)] + d[e, :] + scae[m, e, :]
                l = jnp.where(logit > 0, logit, logit * 0.2)
                ex = jnp.exp(l)
                rows[e, pl.ds(logit_off, 16)] = ex
                for h in range(nheads):
                    rows[e, pl.ds(h * 16, 16)] = rows[e, pl.ds(h * 16, 16)] * ex[h]

            pltpu.sync_copy(rows, shacc.at[scidx.at[1, m]], add=True)

        def step(kk, bufs, obufs, prefetch):
            wait_gather(*bufs)
            compute_scatter(kk, bufs[0], bufs[1])
            if prefetch:
                kn = kk + 1

                @pl.when(lax.rem(kn, CPS) == 0)
                def _():
                    copy_superchunk(lax.div(kn, CPS))

                start_gather(kn, *obufs)

        copy_superchunk(0)
        start_gather(0, rows0, d0, sem0)

        @pl.loop(0, NCH - 1, step=2)
        def _(kk):
            step(kk, (rows0, d0, sem0), (rows1, d1, sem1), True)
            step(kk + 1, (rows1, d1, sem1), (rows0, d0, sem0), True)

        step(NCH - 1, (rows0, d0, sem0), (rows1, d1, sem1), False)

        plsc.subcore_barrier()
        pltpu.sync_copy(shacc.at[pl.ds(s * STRIPE, STRIPE)],
                        accp.at[c, pl.ds(s * STRIPE, STRIPE)])

        @pl.when(s == 0)
        def _():
            pltpu.sync_copy(shacc.at[pl.ds(TAIL_OFF, TAIL)],
                            accp.at[c, pl.ds(TAIL_OFF, TAIL)])

    return k(table, adst, eidx3, ae3, zeros)


def kernel(x, edge_index, edge_attr, W1, att_src1, att_dst1, We1, att_edge1,
           b1, W2, att_src2, att_dst2, We2, att_edge2, b2):
    f32 = jnp.float32
    Wsrc1 = _contract(W1, att_src1, H1, C1)
    Wdst1 = _contract(W1, att_dst1, H1, C1)
    Me1 = _contract(We1, att_edge1, H1, C1)
    Wsrc2 = _contract(W2, att_src2, 1, C2)
    Wdst2 = _contract(W2, att_dst2, 1, C2)
    Me2 = _contract(We2, att_edge2, 1, C2)
    z8 = jnp.zeros((D, 8), f32)
    wcat1 = jnp.concatenate([W1, Wsrc1, z8, Wdst1, z8], axis=1)
    mcat = jnp.concatenate([Me1, jnp.zeros((16, 8), f32),
                            Me2, jnp.zeros((16, 15), f32)], axis=1)
    z15 = jnp.zeros((D, 15), f32)
    wcat2 = jnp.concatenate([W2, Wsrc2, z15, Wdst2, z15], axis=1)

    eidx3 = edge_index.astype(jnp.int32).reshape(2, NCHUNKS, B)

    table1, adst1 = _node_tables(x, wcat1, 144)
    ae1, ae2 = _edge_tables(edge_attr, mcat)
    ae1_3 = ae1.reshape(NCHUNKS, B, 16)
    ae2_3 = ae2.reshape(NCHUNKS, B, 16)

    zeros1 = jnp.zeros((N, 144), f32)
    accp1 = _edge_phase(table1, adst1, eidx3, ae1_3, zeros1,
                        width=144, logit_off=128, nheads=8)

    table2, adst2 = _layer1_finish(accp1, b1, wcat2)

    zeros2 = jnp.zeros((N, 32), f32)
    accp2 = _edge_phase(table2, adst2, eidx3, ae2_3, zeros2,
                        width=32, logit_off=16, nheads=1)

    return _final(accp2, b2)


# speedup vs baseline: 41.3875x; 1.0386x over previous
import functools

import jax
import jax.numpy as jnp
from jax import lax
from jax.experimental import pallas as pl
from jax.experimental.pallas import tpu as pltpu
from jax.experimental.pallas import tpu_sc as plsc

N = 10000
E = 320000
D = 128
H1, C1 = 8, 16
C2 = 16

NC, NS = 2, 16
NW = NC * NS
B1 = 80
B2 = 128
STRIPE = 624
TAIL = N - NS * STRIPE
TAIL_OFF = NS * STRIPE


def _contract(W, att, heads, ch):
    return jnp.einsum('dhc,hc->dh', W.reshape(W.shape[0], heads, ch), att[0])


def _prep_body(x_ref, w_ref, ea_ref, m_ref, t_ref, ad_ref, a1_ref, a2_ref):
    out = jnp.dot(x_ref[...], w_ref[...], preferred_element_type=jnp.float32)
    t_ref[...] = out[:, :144]
    ad_ref[...] = out[:, 144:]
    oe = jnp.dot(ea_ref[...], m_ref[...], preferred_element_type=jnp.float32)
    a1_ref[...] = oe[:, :16]
    a2_ref[...] = oe[:, 16:]


def _prep_tables(x, wcat, edge_attr, mcat):
    nblk, eblk = 400, 12800
    return pl.pallas_call(
        _prep_body,
        grid=(E // eblk,),
        in_specs=[
            pl.BlockSpec((nblk, 128), lambda i: (i, 0)),
            pl.BlockSpec((128, 160), lambda i: (0, 0)),
            pl.BlockSpec((eblk, 16), lambda i: (i, 0)),
            pl.BlockSpec((16, 32), lambda i: (0, 0)),
        ],
        out_specs=[
            pl.BlockSpec((nblk, 144), lambda i: (i, 0)),
            pl.BlockSpec((nblk, 16), lambda i: (i, 0)),
            pl.BlockSpec((eblk, 16), lambda i: (i, 0)),
            pl.BlockSpec((eblk, 16), lambda i: (i, 0)),
        ],
        out_shape=[
            jax.ShapeDtypeStruct((N, 144), jnp.float32),
            jax.ShapeDtypeStruct((N, 16), jnp.float32),
            jax.ShapeDtypeStruct((E, 16), jnp.float32),
            jax.ShapeDtypeStruct((E, 16), jnp.float32),
        ],
    )(x, wcat, edge_attr, mcat)


def _layer1_finish_body(accp_ref, b1_ref, w_ref, t2_ref, ad2_ref):
    acc = accp_ref[0] + accp_ref[1]
    num = acc[:, :128].reshape(-1, H1, C1)
    den = acc[:, 128:136].reshape(-1, H1, 1)
    v = (num / (den + 1e-16)).reshape(-1, 128) + b1_ref[0]
    h1 = jnp.where(v > 0, v, jnp.exp(v) - 1.0)
    out = jnp.dot(h1, w_ref[...], preferred_element_type=jnp.float32)
    t2_ref[...] = out[:, :32]
    ad2_ref[...] = out[:, 32:]


def _layer1_finish(accp1, b1, wcat2):
    blk = 1000
    return pl.pallas_call(
        _layer1_finish_body,
        grid=(N // blk,),
        in_specs=[
            pl.BlockSpec((2, blk, 144), lambda i: (0, i, 0)),
            pl.BlockSpec((1, 128), lambda i: (0, 0)),
            pl.BlockSpec((128, 48), lambda i: (0, 0)),
        ],
        out_specs=[
            pl.BlockSpec((blk, 32), lambda i: (i, 0)),
            pl.BlockSpec((blk, 16), lambda i: (i, 0)),
        ],
        out_shape=[
            jax.ShapeDtypeStruct((N, 32), jnp.float32),
            jax.ShapeDtypeStruct((N, 16), jnp.float32),
        ],
    )(accp1, b1.reshape(1, 128), wcat2)


def _final_body(accp_ref, b2_ref, o_ref):
    acc = accp_ref[0] + accp_ref[1]
    z = acc[:, :16] / (acc[:, 16:17] + 1e-16) + b2_ref[0]
    m = jnp.max(z, axis=1, keepdims=True)
    zz = z - m
    lse = jnp.log(jnp.sum(jnp.exp(zz), axis=1, keepdims=True))
    o_ref[...] = zz - lse


def _final(accp2, b2):
    blk = 1000
    return pl.pallas_call(
        _final_body,
        grid=(N // blk,),
        in_specs=[
            pl.BlockSpec((2, blk, 32), lambda i: (0, i, 0)),
            pl.BlockSpec((1, 16), lambda i: (0, 0)),
        ],
        out_specs=pl.BlockSpec((blk, 16), lambda i: (i, 0)),
        out_shape=jax.ShapeDtypeStruct((N, 16), jnp.float32),
    )(accp2, b2.reshape(1, 16))


def _edge_phase(table, adst, eidx3, ae3, zeros, width, logit_off, nheads,
                Bp, cps, nch, rem_tiles):
    mesh = plsc.VectorSubcoreMesh(core_axis_name="c", subcore_axis_name="s")

    @functools.partial(
        pl.kernel,
        out_type=jax.ShapeDtypeStruct((NC, N, width), jnp.float32),
        mesh=mesh,
        compiler_params=pltpu.CompilerParams(use_tc_tiling_on_sc=False),
        scratch_types=[
            pltpu.VMEM((2, cps, Bp), jnp.int32),
            pltpu.VMEM((cps, Bp, 16), jnp.float32),
            pltpu.VMEM((Bp, width), jnp.float32),
            pltpu.VMEM((Bp, width), jnp.float32),
            pltpu.VMEM((Bp, 16), jnp.float32),
            pltpu.VMEM((Bp, 16), jnp.float32),
            pltpu.VMEM_SHARED((N, width), jnp.float32),
            pltpu.SemaphoreType.DMA,
            pltpu.SemaphoreType.DMA,
        ],
    )
    def k(tab, ad, eidx, aer, zr, accp,
          scidx, scae, rows0, rows1, d0, d1, shacc, sem0, sem1):
        c = lax.axis_index("c")
        s = lax.axis_index("s")
        pltpu.sync_copy(zr.at[pl.ds(s * STRIPE, STRIPE)],
                        shacc.at[pl.ds(s * STRIPE, STRIPE)])

        @pl.when(s == 0)
        def _():
            pltpu.sync_copy(zr.at[pl.ds(TAIL_OFF, TAIL)],
                            shacc.at[pl.ds(TAIL_OFF, TAIL)])

        plsc.subcore_barrier()
        tile = c * NS + s
        chunk0 = tile * nch + jnp.minimum(tile, rem_tiles)

        def copy_superchunk(sc):
            g = chunk0 + sc * cps
            pltpu.sync_copy(eidx.at[:, pl.ds(g, cps)], scidx)
            pltpu.sync_copy(aer.at[pl.ds(g, cps)], scae)

        def start_gather(kk, rows, d, sem):
            m = lax.rem(kk, cps)
            pltpu.async_copy(tab.at[scidx.at[0, m]], rows, sem)
            pltpu.async_copy(ad.at[scidx.at[1, m]], d, sem)

        def wait_gather(rows, d, sem):
            pltpu.make_async_copy(tab.at[scidx.at[0, 0]], rows, sem).wait()
            pltpu.make_async_copy(ad.at[scidx.at[1, 0]], d, sem).wait()

        def compute_scatter(kk, rows, d):
            m = lax.rem(kk, cps)

            @pl.loop(0, Bp)
            def _(e):
                logit = rows[e, pl.ds(logit_off, 16)] + d[e, :] + scae[m, e, :]
                l = jnp.where(logit > 0, logit, logit * 0.2)
                ex = jnp.exp(l)
                rows[e, pl.ds(logit_off, 16)] = ex
                for h in range(nheads):
                    rows[e, pl.ds(h * 16, 16)] = rows[e, pl.ds(h * 16, 16)] * ex[h]

            pltpu.sync_copy(rows, shacc.at[scidx.at[1, m]], add=True)

        def step(kk, bufs, obufs, prefetch):
            wait_gather(*bufs)
            compute_scatter(kk, bufs[0], bufs[1])
            if prefetch:
                kn = kk + 1

                @pl.when(lax.rem(kn, cps) == 0)
                def _():
                    copy_superchunk(lax.div(kn, cps))

                start_gather(kn, *obufs)

        bufs0 = (rows0, d0, sem0)
        bufs1 = (rows1, d1, sem1)
        copy_superchunk(0)
        start_gather(0, rows0, d0, sem0)

        @pl.loop(0, 2 * ((nch - 1) // 2), step=2)
        def _(kk):
            step(kk, bufs0, bufs1, True)
            step(kk + 1, bufs1, bufs0, True)

        if nch % 2:
            step(nch - 1, bufs0, bufs1, False)
        else:
            step(nch - 2, bufs0, bufs1, True)
            step(nch - 1, bufs1, bufs0, False)

        if rem_tiles:
            @pl.when(tile < rem_tiles)
            def _():
                g = chunk0 + nch
                pltpu.sync_copy(eidx.at[:, pl.ds(g, 1)], scidx.at[:, pl.ds(0, 1)])
                pltpu.sync_copy(aer.at[pl.ds(g, 1)], scae.at[pl.ds(0, 1)])
                pltpu.sync_copy(tab.at[scidx.at[0, 0]], rows0)
                pltpu.sync_copy(ad.at[scidx.at[1, 0]], d0)
                compute_scatter(0, rows0, d0)

        plsc.subcore_barrier()
        pltpu.sync_copy(shacc.at[pl.ds(s * STRIPE, STRIPE)],
                        accp.at[c, pl.ds(s * STRIPE, STRIPE)])

        @pl.when(s == 0)
        def _():
            pltpu.sync_copy(shacc.at[pl.ds(TAIL_OFF, TAIL)],
                            accp.at[c, pl.ds(TAIL_OFF, TAIL)])

    return k(table, adst, eidx3, ae3, zeros)


def kernel(x, edge_index, edge_attr, W1, att_src1, att_dst1, We1, att_edge1,
           b1, W2, att_src2, att_dst2, We2, att_edge2, b2):
    f32 = jnp.float32
    Wsrc1 = _contract(W1, att_src1, H1, C1)
    Wdst1 = _contract(W1, att_dst1, H1, C1)
    Me1 = _contract(We1, att_edge1, H1, C1)
    Wsrc2 = _contract(W2, att_src2, 1, C2)
    Wdst2 = _contract(W2, att_dst2, 1, C2)
    Me2 = _contract(We2, att_edge2, 1, C2)
    z8 = jnp.zeros((D, 8), f32)
    wcat1 = jnp.concatenate([W1, Wsrc1, z8, Wdst1, z8], axis=1)
    mcat = jnp.concatenate([Me1, jnp.zeros((16, 8), f32),
                            Me2, jnp.zeros((16, 15), f32)], axis=1)
    z15 = jnp.zeros((D, 15), f32)
    wcat2 = jnp.concatenate([W2, Wsrc2, z15, Wdst2, z15], axis=1)

    ei32 = edge_index.astype(jnp.int32)

    table1, adst1, ae1, ae2 = _prep_tables(x, wcat1, edge_attr, mcat)

    zeros1 = jnp.zeros((N, 144), f32)
    accp1 = _edge_phase(table1, adst1,
                        ei32.reshape(2, E // B1, B1),
                        ae1.reshape(E // B1, B1, 16), zeros1,
                        width=144, logit_off=128, nheads=8,
                        Bp=B1, cps=5, nch=(E // B1) // NW, rem_tiles=0)

    table2, adst2 = _layer1_finish(accp1, b1, wcat2)

    zeros2 = jnp.zeros((N, 32), f32)
    nch2 = (E // B2) // NW
    accp2 = _edge_phase(table2, adst2,
                        ei32.reshape(2, E // B2, B2),
                        ae2.reshape(E // B2, B2, 16), zeros2,
                        width=32, logit_off=16, nheads=1,
                        Bp=B2, cps=6, nch=nch2,
                        rem_tiles=(E // B2) - nch2 * NW)

    return _final(accp2, b2)


# speedup vs baseline: 53.0728x; 1.2823x over previous
import functools

import jax
import jax.numpy as jnp
from jax import lax
from jax.experimental import pallas as pl
from jax.experimental.pallas import tpu as pltpu
from jax.experimental.pallas import tpu_sc as plsc

N = 10000
E = 320000
D = 128
H1, C1 = 8, 16
C2 = 16

NC, NS = 2, 16
NW = NC * NS
B1 = 80
B2 = 128
STRIPE = 624
TAIL = N - NS * STRIPE
TAIL_OFF = NS * STRIPE


def _contract(W, att, heads, ch):
    return jnp.einsum('dhc,hc->dh', W.reshape(W.shape[0], heads, ch), att[0])


def _prep_body(x_ref, w_ref, ea_ref, m_ref, t_ref, ad_ref, a1_ref, a2_ref):
    out = jnp.dot(x_ref[...], w_ref[...], preferred_element_type=jnp.float32)
    t_ref[...] = out[:, :144]
    ad_ref[...] = out[:, 144:]
    oe = jnp.dot(ea_ref[...], m_ref[...], preferred_element_type=jnp.float32)
    a1_ref[...] = oe[:, :16]
    a2_ref[...] = oe[:, 16:]


def _prep_tables(x, wcat, edge_attr, mcat):
    nblk, eblk = 400, 12800
    return pl.pallas_call(
        _prep_body,
        grid=(E // eblk,),
        in_specs=[
            pl.BlockSpec((nblk, 128), lambda i: (i, 0)),
            pl.BlockSpec((128, 160), lambda i: (0, 0)),
            pl.BlockSpec((eblk, 16), lambda i: (i, 0)),
            pl.BlockSpec((16, 32), lambda i: (0, 0)),
        ],
        out_specs=[
            pl.BlockSpec((nblk, 144), lambda i: (i, 0)),
            pl.BlockSpec((nblk, 16), lambda i: (i, 0)),
            pl.BlockSpec((eblk, 16), lambda i: (i, 0)),
            pl.BlockSpec((eblk, 16), lambda i: (i, 0)),
        ],
        out_shape=[
            jax.ShapeDtypeStruct((N, 144), jnp.float32),
            jax.ShapeDtypeStruct((N, 16), jnp.float32),
            jax.ShapeDtypeStruct((E, 16), jnp.float32),
            jax.ShapeDtypeStruct((E, 16), jnp.float32),
        ],
    )(x, wcat, edge_attr, mcat)


def _layer1_finish_body(accp_ref, b1_ref, w_ref, t2_ref, ad2_ref):
    acc = accp_ref[0] + accp_ref[1]
    num = acc[:, :128].reshape(-1, H1, C1)
    den = acc[:, 128:136].reshape(-1, H1, 1)
    v = (num / (den + 1e-16)).reshape(-1, 128) + b1_ref[0]
    h1 = jnp.where(v > 0, v, jnp.exp(v) - 1.0)
    out = jnp.dot(h1, w_ref[...], preferred_element_type=jnp.float32)
    t2_ref[...] = out[:, :32]
    ad2_ref[...] = out[:, 32:]


def _layer1_finish(accp1, b1, wcat2):
    blk = 1000
    return pl.pallas_call(
        _layer1_finish_body,
        grid=(N // blk,),
        in_specs=[
            pl.BlockSpec((2, blk, 144), lambda i: (0, i, 0)),
            pl.BlockSpec((1, 128), lambda i: (0, 0)),
            pl.BlockSpec((128, 48), lambda i: (0, 0)),
        ],
        out_specs=[
            pl.BlockSpec((blk, 32), lambda i: (i, 0)),
            pl.BlockSpec((blk, 16), lambda i: (i, 0)),
        ],
        out_shape=[
            jax.ShapeDtypeStruct((N, 32), jnp.float32),
            jax.ShapeDtypeStruct((N, 16), jnp.float32),
        ],
    )(accp1, b1.reshape(1, 128), wcat2)


def _final_body(accp_ref, b2_ref, o_ref):
    acc = accp_ref[0] + accp_ref[1]
    z = acc[:, :16] / (acc[:, 16:17] + 1e-16) + b2_ref[0]
    m = jnp.max(z, axis=1, keepdims=True)
    zz = z - m
    lse = jnp.log(jnp.sum(jnp.exp(zz), axis=1, keepdims=True))
    o_ref[...] = zz - lse


def _final(accp2, b2):
    blk = 1000
    return pl.pallas_call(
        _final_body,
        grid=(N // blk,),
        in_specs=[
            pl.BlockSpec((2, blk, 32), lambda i: (0, i, 0)),
            pl.BlockSpec((1, 16), lambda i: (0, 0)),
        ],
        out_specs=pl.BlockSpec((blk, 16), lambda i: (i, 0)),
        out_shape=jax.ShapeDtypeStruct((N, 16), jnp.float32),
    )(accp2, b2.reshape(1, 16))


def _edge_phase(table, adst, eidx3, ae3, zeros, width, logit_off, nheads,
                Bp, cps, nch, rem_tiles):
    mesh = plsc.VectorSubcoreMesh(core_axis_name="c", subcore_axis_name="s")

    @functools.partial(
        pl.kernel,
        out_type=jax.ShapeDtypeStruct((NC, N, width), jnp.float32),
        mesh=mesh,
        compiler_params=pltpu.CompilerParams(use_tc_tiling_on_sc=False),
        scratch_types=[
            pltpu.VMEM((2, cps, Bp), jnp.int32),
            pltpu.VMEM((cps, Bp, 16), jnp.float32),
            pltpu.VMEM((Bp, width), jnp.float32),
            pltpu.VMEM((Bp, width), jnp.float32),
            pltpu.VMEM((Bp, 16), jnp.float32),
            pltpu.VMEM((Bp, 16), jnp.float32),
            pltpu.VMEM_SHARED((N, width), jnp.float32),
            pltpu.SemaphoreType.DMA,
            pltpu.SemaphoreType.DMA,
        ],
    )
    def k(tab, ad, eidx, aer, zr, accp,
          scidx, scae, rows0, rows1, d0, d1, shacc, sem0, sem1):
        c = lax.axis_index("c")
        s = lax.axis_index("s")
        pltpu.sync_copy(zr.at[pl.ds(s * STRIPE, STRIPE)],
                        shacc.at[pl.ds(s * STRIPE, STRIPE)])

        @pl.when(s == 0)
        def _():
            pltpu.sync_copy(zr.at[pl.ds(TAIL_OFF, TAIL)],
                            shacc.at[pl.ds(TAIL_OFF, TAIL)])

        plsc.subcore_barrier()
        tile = c * NS + s
        chunk0 = tile * nch + jnp.minimum(tile, rem_tiles)

        def copy_superchunk(sc):
            g = chunk0 + sc * cps
            pltpu.sync_copy(eidx.at[:, pl.ds(g, cps)], scidx)
            pltpu.sync_copy(aer.at[pl.ds(g, cps)], scae)

        def start_gather(kk, rows, d, sem):
            m = lax.rem(kk, cps)
            pltpu.async_copy(tab.at[scidx.at[0, m]], rows, sem)
            pltpu.async_copy(ad.at[scidx.at[1, m]], d, sem)

        def wait_gather(rows, d, sem):
            pltpu.make_async_copy(tab.at[scidx.at[0, 0]], rows, sem).wait()
            pltpu.make_async_copy(ad.at[scidx.at[1, 0]], d, sem).wait()

        def compute_scatter(kk, rows, d):
            m = lax.rem(kk, cps)

            @plsc.parallel_loop(0, Bp, unroll=4)
            def _(e):
                logit = rows[e, pl.ds(logit_off, 16)] + d[e, :] + scae[m, e, :]
                l = jnp.where(logit > 0, logit, logit * 0.2)
                ex = jnp.exp(l)
                rows[e, pl.ds(logit_off, 16)] = ex
                for h in range(nheads):
                    rows[e, pl.ds(h * 16, 16)] = rows[e, pl.ds(h * 16, 16)] * ex[h]

            pltpu.sync_copy(rows, shacc.at[scidx.at[1, m]], add=True)

        def step(kk, bufs, obufs, prefetch):
            wait_gather(*bufs)
            compute_scatter(kk, bufs[0], bufs[1])
            if prefetch:
                kn = kk + 1

                @pl.when(lax.rem(kn, cps) == 0)
                def _():
                    copy_superchunk(lax.div(kn, cps))

                start_gather(kn, *obufs)

        bufs0 = (rows0, d0, sem0)
        bufs1 = (rows1, d1, sem1)
        copy_superchunk(0)
        start_gather(0, rows0, d0, sem0)

        @pl.loop(0, 2 * ((nch - 1) // 2), step=2)
        def _(kk):
            step(kk, bufs0, bufs1, True)
            step(kk + 1, bufs1, bufs0, True)

        if nch % 2:
            step(nch - 1, bufs0, bufs1, False)
        else:
            step(nch - 2, bufs0, bufs1, True)
            step(nch - 1, bufs1, bufs0, False)

        if rem_tiles:
            @pl.when(tile < rem_tiles)
            def _():
                g = chunk0 + nch
                pltpu.sync_copy(eidx.at[:, pl.ds(g, 1)], scidx.at[:, pl.ds(0, 1)])
                pltpu.sync_copy(aer.at[pl.ds(g, 1)], scae.at[pl.ds(0, 1)])
                pltpu.sync_copy(tab.at[scidx.at[0, 0]], rows0)
                pltpu.sync_copy(ad.at[scidx.at[1, 0]], d0)
                compute_scatter(0, rows0, d0)

        plsc.subcore_barrier()
        pltpu.sync_copy(shacc.at[pl.ds(s * STRIPE, STRIPE)],
                        accp.at[c, pl.ds(s * STRIPE, STRIPE)])

        @pl.when(s == 0)
        def _():
            pltpu.sync_copy(shacc.at[pl.ds(TAIL_OFF, TAIL)],
                            accp.at[c, pl.ds(TAIL_OFF, TAIL)])

    return k(table, adst, eidx3, ae3, zeros)


def kernel(x, edge_index, edge_attr, W1, att_src1, att_dst1, We1, att_edge1,
           b1, W2, att_src2, att_dst2, We2, att_edge2, b2):
    f32 = jnp.float32
    Wsrc1 = _contract(W1, att_src1, H1, C1)
    Wdst1 = _contract(W1, att_dst1, H1, C1)
    Me1 = _contract(We1, att_edge1, H1, C1)
    Wsrc2 = _contract(W2, att_src2, 1, C2)
    Wdst2 = _contract(W2, att_dst2, 1, C2)
    Me2 = _contract(We2, att_edge2, 1, C2)
    z8 = jnp.zeros((D, 8), f32)
    wcat1 = jnp.concatenate([W1, Wsrc1, z8, Wdst1, z8], axis=1)
    mcat = jnp.concatenate([Me1, jnp.zeros((16, 8), f32),
                            Me2, jnp.zeros((16, 15), f32)], axis=1)
    z15 = jnp.zeros((D, 15), f32)
    wcat2 = jnp.concatenate([W2, Wsrc2, z15, Wdst2, z15], axis=1)

    ei32 = edge_index.astype(jnp.int32)

    table1, adst1, ae1, ae2 = _prep_tables(x, wcat1, edge_attr, mcat)

    zeros1 = jnp.zeros((N, 144), f32)
    accp1 = _edge_phase(table1, adst1,
                        ei32.reshape(2, E // B1, B1),
                        ae1.reshape(E // B1, B1, 16), zeros1,
                        width=144, logit_off=128, nheads=8,
                        Bp=B1, cps=5, nch=(E // B1) // NW, rem_tiles=0)

    table2, adst2 = _layer1_finish(accp1, b1, wcat2)

    zeros2 = jnp.zeros((N, 32), f32)
    nch2 = (E // B2) // NW
    accp2 = _edge_phase(table2, adst2,
                        ei32.reshape(2, E // B2, B2),
                        ae2.reshape(E // B2, B2, 16), zeros2,
                        width=32, logit_off=16, nheads=1,
                        Bp=B2, cps=6, nch=nch2,
                        rem_tiles=(E // B2) - nch2 * NW)

    return _final(accp2, b2)
